# Initial kernel scaffold; baseline (speedup 1.0000x reference)
import jax
import jax.numpy as jnp
from jax.experimental import pallas as pl


def kernel(points, batch_size):
    raise NotImplementedError("write your pallas kernel here")


# speedup vs baseline: 9.9948x; 9.9948x over previous
import jax
import jax.numpy as jnp
from jax.experimental import pallas as pl
from jax.experimental.pallas import tpu as pltpu

_B = 4
_NKP = 2048
_R = 128
_C = 128


def _fps_body(x_ref, y_ref, z_ref, pts_ref, kp_ref, kf_ref):
    x = x_ref[0]
    y = y_ref[0]
    z = z_ref[0]
    n = _R * _C

    row_i = jax.lax.broadcasted_iota(jnp.int32, (_R, _C), 0)
    col_i = jax.lax.broadcasted_iota(jnp.int32, (_R, _C), 1)
    lin = row_i * _C + col_i

    row0 = pts_ref[0, 0:1, :]
    kp_ref[0, 0:1, :] = row0[:, 0:3]
    kf_ref[0, 0:1, :] = row0[:, 3:4]

    def step(i, carry):
        dist, lx, ly, lz = carry
        dx = x - lx
        dy = y - ly
        dz = z - lz
        d = (dx * dx + dz * dz) + dy * dy
        dist = jnp.minimum(dist, d)
        m = jnp.max(dist)
        j = jnp.min(jnp.where(dist == m, lin, jnp.int32(n)))
        row = pts_ref[0, pl.ds(j, 1), :]
        kp_ref[0, pl.ds(i, 1), :] = row[:, 0:3]
        kf_ref[0, pl.ds(i, 1), :] = row[:, 3:4]
        return dist, row[0, 0], row[0, 1], row[0, 2]

    dist0 = jnp.full((_R, _C), 1e10, dtype=jnp.float32)
    jax.lax.fori_loop(
        1, _NKP, step, (dist0, row0[0, 0], row0[0, 1], row0[0, 2])
    )


def kernel(points, batch_size):
    del batch_size
    n_total = points.shape[0]
    n = n_total // _B
    pts = points.reshape(_B, n, points.shape[1])
    xyzf = pts[:, :, 1:5]
    x = xyzf[:, :, 0].reshape(_B, _R, _C)
    y = xyzf[:, :, 1].reshape(_B, _R, _C)
    z = xyzf[:, :, 2].reshape(_B, _R, _C)

    kp, kf = pl.pallas_call(
        _fps_body,
        grid=(_B,),
        in_specs=[
            pl.BlockSpec((1, _R, _C), lambda b: (b, 0, 0)),
            pl.BlockSpec((1, _R, _C), lambda b: (b, 0, 0)),
            pl.BlockSpec((1, _R, _C), lambda b: (b, 0, 0)),
            pl.BlockSpec((1, n, 4), lambda b: (b, 0, 0)),
        ],
        out_specs=[
            pl.BlockSpec((1, _NKP, 3), lambda b: (b, 0, 0)),
            pl.BlockSpec((1, _NKP, 1), lambda b: (b, 0, 0)),
        ],
        out_shape=[
            jax.ShapeDtypeStruct((_B, _NKP, 3), jnp.float32),
            jax.ShapeDtypeStruct((_B, _NKP, 1), jnp.float32),
        ],
        compiler_params=pltpu.CompilerParams(
            dimension_semantics=("parallel",),
        ),
    )(x, y, z, xyzf)
    return kp, kf


# speedup vs baseline: 11.7655x; 1.1772x over previous
import jax
import jax.numpy as jnp
from jax.experimental import pallas as pl
from jax.experimental.pallas import tpu as pltpu

_B = 4
_PER_STEP = 2
_NKP = 2048
_R = 128
_C = 128


def _fps_body(x_ref, y_ref, z_ref, pts_ref, kp_ref, kf_ref):
    n = _R * _C
    P = _PER_STEP

    xs = [x_ref[p] for p in range(P)]
    ys = [y_ref[p] for p in range(P)]
    zs = [z_ref[p] for p in range(P)]

    row_i = jax.lax.broadcasted_iota(jnp.int32, (_R, _C), 0)
    col_i = jax.lax.broadcasted_iota(jnp.int32, (_R, _C), 1)
    lin = row_i * _C + col_i

    rows0 = []
    for p in range(P):
        r0 = pts_ref[p, 0:1, :]
        kp_ref[p, 0:1, :] = r0[:, 0:3]
        kf_ref[p, 0:1, :] = r0[:, 3:4]
        rows0.append(r0)

    def step(i, carry):
        dists, lxs, lys, lzs = carry
        new_dists = []
        new_l = []
        for p in range(P):
            dx = xs[p] - lxs[p]
            dy = ys[p] - lys[p]
            dz = zs[p] - lzs[p]
            d = (dx * dx + dz * dz) + dy * dy
            new_dists.append(jnp.minimum(dists[p], d))
        for p in range(P):
            dist = new_dists[p]
            m = jnp.max(dist)
            j = jnp.min(jnp.where(dist == m, lin, jnp.int32(n)))
            row = pts_ref[p, pl.ds(j, 1), :]
            kp_ref[p, pl.ds(i, 1), :] = row[:, 0:3]
            kf_ref[p, pl.ds(i, 1), :] = row[:, 3:4]
            new_l.append((row[0, 0], row[0, 1], row[0, 2]))
        return (
            tuple(new_dists),
            tuple(r[0] for r in new_l),
            tuple(r[1] for r in new_l),
            tuple(r[2] for r in new_l),
        )

    dist0 = jnp.full((_R, _C), 1e10, dtype=jnp.float32)
    jax.lax.fori_loop(
        1,
        _NKP,
        step,
        (
            tuple(dist0 for _ in range(P)),
            tuple(r[0, 0] for r in rows0),
            tuple(r[0, 1] for r in rows0),
            tuple(r[0, 2] for r in rows0),
        ),
    )


def kernel(points, batch_size):
    del batch_size
    n_total = points.shape[0]
    n = n_total // _B
    pts = points.reshape(_B, n, points.shape[1])
    xyzf = pts[:, :, 1:5]
    x = xyzf[:, :, 0].reshape(_B, _R, _C)
    y = xyzf[:, :, 1].reshape(_B, _R, _C)
    z = xyzf[:, :, 2].reshape(_B, _R, _C)

    grid = (_B // _PER_STEP,)
    kp, kf = pl.pallas_call(
        _fps_body,
        grid=grid,
        in_specs=[
            pl.BlockSpec((_PER_STEP, _R, _C), lambda b: (b, 0, 0)),
            pl.BlockSpec((_PER_STEP, _R, _C), lambda b: (b, 0, 0)),
            pl.BlockSpec((_PER_STEP, _R, _C), lambda b: (b, 0, 0)),
            pl.BlockSpec((_PER_STEP, n, 4), lambda b: (b, 0, 0)),
        ],
        out_specs=[
            pl.BlockSpec((_PER_STEP, _NKP, 3), lambda b: (b, 0, 0)),
            pl.BlockSpec((_PER_STEP, _NKP, 1), lambda b: (b, 0, 0)),
        ],
        out_shape=[
            jax.ShapeDtypeStruct((_B, _NKP, 3), jnp.float32),
            jax.ShapeDtypeStruct((_B, _NKP, 1), jnp.float32),
        ],
        compiler_params=pltpu.CompilerParams(
            dimension_semantics=("parallel",),
        ),
    )(x, y, z, xyzf)
    return kp, kf
